# baseline (speedup 1.0000x reference)
import jax
import jax.numpy as jnp
from jax.experimental import pallas as pl
from jax.experimental.pallas import tpu as pltpu

_N_EXPERTS = 8
_D_MODEL = 768
_N_TOKENS = 2048
_FB = 128
_K = _D_MODEL // _FB


def _moe_kernel(x_ref, wg_ref, we_ref, be_ref, out_ref, gw_ref, xb_ref):
    @pl.when(pl.program_id(0) == 0)
    def _prologue():
        x = x_ref[...]
        logits = jax.lax.dot_general(
            x, wg_ref[...], (((1,), (1,)), ((), ())),
            preferred_element_type=jnp.float32)
        g = jax.nn.softmax(logits, axis=1)
        e_iota = jax.lax.broadcasted_iota(
            jnp.int32, (_N_TOKENS, _N_EXPERTS), 1)
        m1 = jnp.max(g, axis=1, keepdims=True)
        i1 = jnp.min(jnp.where(g == m1, e_iota, _N_EXPERTS), axis=1,
                     keepdims=True)
        g2 = jnp.where(e_iota == i1, -jnp.inf, g)
        m2 = jnp.max(g2, axis=1, keepdims=True)
        i2 = jnp.min(jnp.where(g2 == m2, e_iota, _N_EXPERTS), axis=1,
                     keepdims=True)
        gw_ref[...] = jnp.where((e_iota == i1) | (e_iota == i2), g, 0.0)
        xb_ref[...] = x.astype(jnp.bfloat16)

    gw = gw_ref[...]
    xb = xb_ref[...]
    be = be_ref[...]
    acc = jnp.zeros((_N_TOKENS, _FB), jnp.float32)
    for e in range(_N_EXPERTS):
        ye = jax.lax.dot_general(
            xb, we_ref[e].astype(jnp.bfloat16), (((1,), (1,)), ((), ())),
            preferred_element_type=jnp.float32)
        acc = acc + gw[:, e][:, None] * (ye + be[e][None, :])
    out_ref[...] = acc


def kernel(input_data, W_gate, W_experts, b_experts):
    return pl.pallas_call(
        _moe_kernel,
        grid=(_K,),
        in_specs=[
            pl.BlockSpec((_N_TOKENS, _D_MODEL), lambda k: (0, 0)),
            pl.BlockSpec((_N_EXPERTS, _D_MODEL), lambda k: (0, 0)),
            pl.BlockSpec((_N_EXPERTS, _FB, _D_MODEL), lambda k: (0, k, 0)),
            pl.BlockSpec((_N_EXPERTS, _FB), lambda k: (0, k)),
        ],
        out_specs=pl.BlockSpec((_N_TOKENS, _FB), lambda k: (0, k)),
        out_shape=jax.ShapeDtypeStruct((_N_TOKENS, _D_MODEL), jnp.float32),
        scratch_shapes=[
            pltpu.VMEM((_N_TOKENS, _N_EXPERTS), jnp.float32),
            pltpu.VMEM((_N_TOKENS, _D_MODEL), jnp.bfloat16),
        ],
    )(input_data, W_gate, W_experts, b_experts)


# speedup vs baseline: 1.6049x; 1.6049x over previous
import jax
import jax.numpy as jnp
from jax.experimental import pallas as pl
from jax.experimental.pallas import tpu as pltpu

_N_EXPERTS = 8
_D_MODEL = 768
_N_TOKENS = 2048
_FB = 256
_K = _D_MODEL // _FB


def _moe_kernel(x_ref, wg_ref, we_ref, be_ref, out_ref, gw_ref, xb_ref):
    @pl.when(pl.program_id(0) == 0)
    def _prologue():
        x = x_ref[...]
        logits = jax.lax.dot_general(
            x, wg_ref[...], (((1,), (1,)), ((), ())),
            preferred_element_type=jnp.float32)
        g = jax.nn.softmax(logits, axis=1)
        e_iota = jax.lax.broadcasted_iota(
            jnp.int32, (_N_TOKENS, _N_EXPERTS), 1)
        m1 = jnp.max(g, axis=1, keepdims=True)
        i1 = jnp.min(jnp.where(g == m1, e_iota, _N_EXPERTS), axis=1,
                     keepdims=True)
        g2 = jnp.where(e_iota == i1, -jnp.inf, g)
        m2 = jnp.max(g2, axis=1, keepdims=True)
        i2 = jnp.min(jnp.where(g2 == m2, e_iota, _N_EXPERTS), axis=1,
                     keepdims=True)
        gw_ref[...] = jnp.where((e_iota == i1) | (e_iota == i2), g, 0.0)
        xb_ref[...] = x.astype(jnp.bfloat16)

    gw = gw_ref[...]
    xb = xb_ref[...]
    be = be_ref[...]
    acc = jnp.zeros((_N_TOKENS, _FB), jnp.float32)
    for e in range(_N_EXPERTS):
        ye = jax.lax.dot_general(
            xb, we_ref[e].astype(jnp.bfloat16), (((1,), (1,)), ((), ())),
            preferred_element_type=jnp.float32)
        acc = acc + gw[:, e][:, None] * (ye + be[e][None, :])
    out_ref[...] = acc


def kernel(input_data, W_gate, W_experts, b_experts):
    return pl.pallas_call(
        _moe_kernel,
        grid=(_K,),
        in_specs=[
            pl.BlockSpec((_N_TOKENS, _D_MODEL), lambda k: (0, 0)),
            pl.BlockSpec((_N_EXPERTS, _D_MODEL), lambda k: (0, 0)),
            pl.BlockSpec((_N_EXPERTS, _FB, _D_MODEL), lambda k: (0, k, 0)),
            pl.BlockSpec((_N_EXPERTS, _FB), lambda k: (0, k)),
        ],
        out_specs=pl.BlockSpec((_N_TOKENS, _FB), lambda k: (0, k)),
        out_shape=jax.ShapeDtypeStruct((_N_TOKENS, _D_MODEL), jnp.float32),
        scratch_shapes=[
            pltpu.VMEM((_N_TOKENS, _N_EXPERTS), jnp.float32),
            pltpu.VMEM((_N_TOKENS, _D_MODEL), jnp.bfloat16),
        ],
    )(input_data, W_gate, W_experts, b_experts)
